# Initial kernel scaffold; baseline (speedup 1.0000x reference)
import jax
import jax.numpy as jnp
from jax.experimental import pallas as pl


def kernel(x, pos, edge_index, batch, W1, W2, W3, W4, Wfc):
    raise NotImplementedError("write your pallas kernel here")


# speedup vs baseline: 11.7152x; 11.7152x over previous
import functools

import jax
import jax.numpy as jnp
from jax import lax
from jax.experimental import pallas as pl
from jax.experimental.pallas import tpu as pltpu
from jax.experimental.pallas import tpu_sc as plsc

NN = 100000
EE = 1600000
NB = 4
NGRID = 64
FPOOL = 32
NOUT = 2

NSC = 2
NTS = 16
LANES = 16

KE = 1000
CH = 5000
NCCH = NN // CH


def _mesh():
    return plsc.VectorSubcoreMesh(core_axis_name="c", subcore_axis_name="s")


ZROWS = 6240
ZTAIL = NN - NTS * ZROWS
_ZCH = tuple((i * KE, KE) for i in range(ZROWS // KE)) + (
    ((ZROWS // KE) * KE, ZROWS % KE),)


def _zero_acc2d(rows_v, acc, s):
    def zv(i, carry):
        rows_v[i] = jnp.zeros((LANES,), jnp.float32)
        return carry

    lax.fori_loop(0, KE, zv, 0)
    for off, sz in _ZCH:
        pltpu.sync_copy(rows_v.at[pl.ds(0, sz)],
                        acc.at[pl.ds(s * ZROWS + off, sz)])

    @pl.when(s == 0)
    def _():
        pltpu.sync_copy(rows_v.at[pl.ds(0, ZTAIL)],
                        acc.at[pl.ds(NTS * ZROWS, ZTAIL)])


def _writeback2d(rows_v, acc, out_hbm, c, s):
    for off, sz in _ZCH:
        pltpu.sync_copy(acc.at[pl.ds(s * ZROWS + off, sz)],
                        rows_v.at[pl.ds(0, sz)])
        pltpu.sync_copy(rows_v.at[pl.ds(0, sz)],
                        out_hbm.at[pl.ds(c * NN + s * ZROWS + off, sz)])

    @pl.when(s == 0)
    def _():
        pltpu.sync_copy(acc.at[pl.ds(NTS * ZROWS, ZTAIL)],
                        rows_v.at[pl.ds(0, ZTAIL)])
        pltpu.sync_copy(rows_v.at[pl.ds(0, ZTAIL)],
                        out_hbm.at[pl.ds(c * NN + NTS * ZROWS, ZTAIL)])


def _sc_agg_f1(xf, src, dst):
    epc = EE // NSC
    ept = epc // NTS
    nch = ept // KE
    g_per = (KE + LANES - 1) // LANES
    zrows = 6240
    tail = NN - NTS * zrows

    @functools.partial(
        pl.kernel,
        out_type=jax.ShapeDtypeStruct((2 * NN,), jnp.float32),
        mesh=_mesh(),
        compiler_params=pltpu.CompilerParams(needs_layout_passes=False, use_tc_tiling_on_sc=False),
        scratch_types=[
            pltpu.VMEM((NN,), jnp.float32),
            pltpu.VMEM((KE + LANES,), jnp.int32),
            pltpu.VMEM((KE,), jnp.int32),
            pltpu.VMEM((KE + LANES,), jnp.float32),
            pltpu.VMEM_SHARED((NN,), jnp.float32),
            pltpu.SemaphoreType.DMA,
        ],
    )
    def k(x_hbm, src_hbm, dst_hbm, out_hbm,
          xbuf, src_v, dst_v, val_v, acc, sem):
        c = lax.axis_index("c")
        s = lax.axis_index("s")

        def zv(g, carry):
            val_v[pl.ds(g * LANES, LANES)] = jnp.zeros((LANES,), jnp.float32)
            return carry

        lax.fori_loop(0, (KE + LANES) // LANES, zv, 0)
        src_v[pl.ds(KE, LANES)] = jnp.zeros((LANES,), jnp.int32)
        for off, sz in _ZCH:
            pltpu.sync_copy(val_v.at[pl.ds(0, sz)],
                            acc.at[pl.ds(s * zrows + off, sz)])

        @pl.when(s == 0)
        def _():
            pltpu.sync_copy(val_v.at[pl.ds(0, tail)],
                            acc.at[pl.ds(NTS * zrows, tail)])

        pltpu.sync_copy(x_hbm, xbuf)
        plsc.subcore_barrier()
        ebase = c * epc + s * ept

        def chunk(j, carry):
            eb = ebase + j * KE
            pltpu.sync_copy(src_hbm.at[pl.ds(eb, KE)], src_v.at[pl.ds(0, KE)])
            pltpu.sync_copy(dst_hbm.at[pl.ds(eb, KE)], dst_v)

            def lane(g, carry2):
                sv = src_v[pl.ds(g * LANES, LANES)]
                val_v[pl.ds(g * LANES, LANES)] = plsc.load_gather(xbuf, [sv])
                return carry2

            lax.fori_loop(0, g_per, lane, 0)
            pltpu.sync_copy(val_v.at[pl.ds(0, KE)], acc.at[dst_v], add=True)
            return carry

        lax.fori_loop(0, nch, chunk, 0)
        plsc.subcore_barrier()
        for off, sz in _ZCH:
            pltpu.sync_copy(acc.at[pl.ds(s * zrows + off, sz)],
                            val_v.at[pl.ds(0, sz)])
            pltpu.sync_copy(val_v.at[pl.ds(0, sz)],
                            out_hbm.at[pl.ds(c * NN + s * zrows + off, sz)])

        @pl.when(s == 0)
        def _():
            pltpu.sync_copy(acc.at[pl.ds(NTS * zrows, tail)],
                            val_v.at[pl.ds(0, tail)])
            pltpu.sync_copy(val_v.at[pl.ds(0, tail)],
                            out_hbm.at[pl.ds(c * NN + NTS * zrows, tail)])

    return k(xf, src, dst)


def _sc_agg_f16(h, src, dst):
    epc = EE // NSC
    ept = epc // NTS
    nch = ept // KE

    @functools.partial(
        pl.kernel,
        out_type=jax.ShapeDtypeStruct((2 * NN, 16), jnp.float32),
        mesh=_mesh(),
        compiler_params=pltpu.CompilerParams(needs_layout_passes=False, use_tc_tiling_on_sc=False),
        scratch_types=[
            pltpu.VMEM((KE,), jnp.int32),
            pltpu.VMEM((KE,), jnp.int32),
            pltpu.VMEM((KE, 16), jnp.float32),
            pltpu.VMEM_SHARED((NN, 16), jnp.float32),
            pltpu.SemaphoreType.DMA,
        ],
    )
    def k(h_hbm, src_hbm, dst_hbm, out_hbm,
          src_v, dst_v, rows_v, acc, sem):
        c = lax.axis_index("c")
        s = lax.axis_index("s")
        _zero_acc2d(rows_v, acc, s)
        plsc.subcore_barrier()
        ebase = c * epc + s * ept

        def chunk(j, carry):
            eb = ebase + j * KE
            pltpu.sync_copy(src_hbm.at[pl.ds(eb, KE)], src_v)
            pltpu.sync_copy(dst_hbm.at[pl.ds(eb, KE)], dst_v)
            pltpu.async_copy(h_hbm.at[src_v], rows_v, sem).wait()
            pltpu.sync_copy(rows_v, acc.at[dst_v], add=True)
            return carry

        lax.fori_loop(0, nch, chunk, 0)
        plsc.subcore_barrier()
        _writeback2d(rows_v, acc, out_hbm, c, s)

    return k(h, src, dst)


def _sc_agg_f32(h_lo, h_hi, src, dst):
    ept = EE // NTS
    nch = ept // KE

    @functools.partial(
        pl.kernel,
        out_type=jax.ShapeDtypeStruct((2 * NN, 16), jnp.float32),
        mesh=_mesh(),
        compiler_params=pltpu.CompilerParams(needs_layout_passes=False, use_tc_tiling_on_sc=False),
        scratch_types=[
            pltpu.VMEM((KE,), jnp.int32),
            pltpu.VMEM((KE,), jnp.int32),
            pltpu.VMEM((KE, 16), jnp.float32),
            pltpu.VMEM_SHARED((NN, 16), jnp.float32),
            pltpu.SemaphoreType.DMA,
        ],
    )
    def k(hlo_hbm, hhi_hbm, src_hbm, dst_hbm, out_hbm,
          src_v, dst_v, rows_v, acc, sem):
        c = lax.axis_index("c")
        s = lax.axis_index("s")
        _zero_acc2d(rows_v, acc, s)
        plsc.subcore_barrier()
        ebase = s * ept

        def chunk(j, carry):
            eb = ebase + j * KE
            pltpu.sync_copy(src_hbm.at[pl.ds(eb, KE)], src_v)
            pltpu.sync_copy(dst_hbm.at[pl.ds(eb, KE)], dst_v)

            @pl.when(c == 0)
            def _():
                pltpu.async_copy(hlo_hbm.at[src_v], rows_v, sem).wait()

            @pl.when(c == 1)
            def _():
                pltpu.async_copy(hhi_hbm.at[src_v], rows_v, sem).wait()

            pltpu.sync_copy(rows_v, acc.at[dst_v], add=True)
            return carry

        lax.fori_loop(0, nch, chunk, 0)
        plsc.subcore_barrier()
        _writeback2d(rows_v, acc, out_hbm, c, s)

    return k(h_lo, h_hi, src, dst)


def _sc_pool(h_lo, h_hi, idxq):
    zrows = 6240
    tail = NN - NTS * zrows
    kn = zrows + tail
    nseg = NB * NGRID

    @functools.partial(
        pl.kernel,
        out_type=(jax.ShapeDtypeStruct((nseg, 16), jnp.float32),
                  jax.ShapeDtypeStruct((nseg, 16), jnp.float32)),
        mesh=_mesh(),
        compiler_params=pltpu.CompilerParams(needs_layout_passes=False, use_tc_tiling_on_sc=False),
        scratch_types=[
            pltpu.VMEM((kn, 16), jnp.float32),
            pltpu.VMEM((kn + LANES,), jnp.int32),
            pltpu.VMEM((nseg, 16), jnp.float32),
            pltpu.VMEM((16, 16), jnp.float32),
            pltpu.VMEM((16, 16), jnp.float32),
            pltpu.VMEM_SHARED((NTS * nseg, 16), jnp.float32),
        ],
    )
    def k(hlo_hbm, hhi_hbm, idx_hbm, plo_hbm, phi_hbm,
          hbuf, ibuf, acc, macc, tbuf, stage):
        c = lax.axis_index("c")
        s = lax.axis_index("s")

        def zacc(i, carry):
            acc[i] = jnp.zeros((16,), jnp.float32)
            return carry

        lax.fori_loop(0, nseg, zacc, 0)

        nbase = s * zrows

        @pl.when(c == 0)
        def _():
            pltpu.sync_copy(hlo_hbm.at[pl.ds(nbase, zrows)],
                            hbuf.at[pl.ds(0, zrows)])

        @pl.when(c == 1)
        def _():
            pltpu.sync_copy(hhi_hbm.at[pl.ds(nbase, zrows)],
                            hbuf.at[pl.ds(0, zrows)])

        pltpu.sync_copy(idx_hbm.at[pl.ds(nbase, zrows)],
                        ibuf.at[pl.ds(0, zrows)])

        @pl.when(jnp.logical_and(s == 0, c == 0))
        def _():
            pltpu.sync_copy(hlo_hbm.at[pl.ds(NTS * zrows, tail)],
                            hbuf.at[pl.ds(zrows, tail)])

        @pl.when(jnp.logical_and(s == 0, c == 1))
        def _():
            pltpu.sync_copy(hhi_hbm.at[pl.ds(NTS * zrows, tail)],
                            hbuf.at[pl.ds(zrows, tail)])

        @pl.when(s == 0)
        def _():
            pltpu.sync_copy(idx_hbm.at[pl.ds(NTS * zrows, tail)],
                            ibuf.at[pl.ds(zrows, tail)])

        count = jnp.where(s == 0, zrows + tail, zrows)

        def node(n, carry2):
            i = ibuf[pl.ds(n, LANES)][0]
            acc[i] = jnp.maximum(acc[i], hbuf[n])
            return carry2

        lax.fori_loop(0, count, node, 0)
        pltpu.sync_copy(acc, stage.at[pl.ds(s * nseg, nseg)])
        plsc.subcore_barrier()

        def zm(i, carry):
            macc[i] = jnp.zeros((16,), jnp.float32)
            return carry

        lax.fori_loop(0, 16, zm, 0)
        for t in range(NTS):
            pltpu.sync_copy(stage.at[pl.ds(t * nseg + s * 16, 16)], tbuf)

            def mx(r, carry):
                macc[r] = jnp.maximum(macc[r], tbuf[r])
                return carry

            lax.fori_loop(0, 16, mx, 0)

        @pl.when(c == 0)
        def _():
            pltpu.sync_copy(macc, plo_hbm.at[pl.ds(s * 16, 16)])

        @pl.when(c == 1)
        def _():
            pltpu.sync_copy(macc, phi_hbm.at[pl.ds(s * 16, 16)])

    return k(h_lo, h_hi, idxq)


def _dense1(agg2n, w1):
    def body(a_ref, b_ref, w_ref, o_ref, ssum, ssq):
        p = pl.program_id(0)
        cch = pl.program_id(1)
        pblk = (a_ref[...] + b_ref[...]) * w_ref[...]

        @pl.when(jnp.logical_and(p == 0, cch == 0))
        def _():
            ssum[...] = jnp.zeros_like(ssum)
            ssq[...] = jnp.zeros_like(ssq)

        @pl.when(p == 0)
        def _():
            ssum[...] += jnp.sum(pblk, axis=0, keepdims=True)
            ssq[...] += jnp.sum(pblk * pblk, axis=0, keepdims=True)

        @pl.when(p == 1)
        def _():
            mu = ssum[...] * (1.0 / NN)
            var = ssq[...] * (1.0 / NN) - mu * mu
            o_ref[...] = jnp.maximum((pblk - mu) * lax.rsqrt(var + 1e-5), 0.0)

    return pl.pallas_call(
        body,
        grid=(2, NCCH),
        compiler_params=pltpu.CompilerParams(vmem_limit_bytes=100 * 2**20),
        in_specs=[
            pl.BlockSpec((CH, 1), lambda p, c: (c, 0)),
            pl.BlockSpec((CH, 1), lambda p, c: (NCCH + c, 0)),
            pl.BlockSpec((1, 16), lambda p, c: (0, 0)),
        ],
        out_specs=pl.BlockSpec((CH, 16), lambda p, c: (c, 0)),
        out_shape=jax.ShapeDtypeStruct((NN, 16), jnp.float32),
        scratch_shapes=[pltpu.VMEM((1, 16), jnp.float32),
                        pltpu.VMEM((1, 16), jnp.float32)],
    )(agg2n, agg2n, w1)


def _dense_mm(aggcat, w_a, w_b, pos2d=None, batch2d=None):
    last = pos2d is not None

    def body(*refs):
        if last:
            (a_ref, b_ref, wa_ref, wb_ref, pos_ref, bat_ref,
             lo_ref, hi_ref, idx_ref, ssum, ssq) = refs
        else:
            a_ref, b_ref, wa_ref, wb_ref, lo_ref, hi_ref, ssum, ssq = refs
        p = pl.program_id(0)
        cch = pl.program_id(1)
        pblk = (jnp.dot(a_ref[...], wa_ref[...],
                        preferred_element_type=jnp.float32) +
                jnp.dot(b_ref[...], wb_ref[...],
                        preferred_element_type=jnp.float32))

        @pl.when(jnp.logical_and(p == 0, cch == 0))
        def _():
            ssum[...] = jnp.zeros_like(ssum)
            ssq[...] = jnp.zeros_like(ssq)

        @pl.when(p == 0)
        def _():
            ssum[...] += jnp.sum(pblk, axis=0, keepdims=True)
            ssq[...] += jnp.sum(pblk * pblk, axis=0, keepdims=True)

        @pl.when(p == 1)
        def _():
            mu = ssum[...] * (1.0 / NN)
            var = ssq[...] * (1.0 / NN) - mu * mu
            hn = jnp.maximum((pblk - mu) * lax.rsqrt(var + 1e-5), 0.0)
            lo_ref[...] = hn[:, :16]
            hi_ref[...] = hn[:, 16:]
            if last:
                px = pos_ref[...][:, 0:1]
                py = pos_ref[...][:, 1:2]
                vx = jnp.clip(jnp.floor(px / 15.0).astype(jnp.int32), 0, 7)
                vy = jnp.clip(jnp.floor(py / 12.0).astype(jnp.int32), 0, 7)
                idx_ref[...] = bat_ref[...] * NGRID + vx * 8 + vy

    in_specs = [
        pl.BlockSpec((CH, 16), lambda p, c: (c, 0)),
        pl.BlockSpec((CH, 16), lambda p, c: (NCCH + c, 0)),
        pl.BlockSpec((16, 32), lambda p, c: (0, 0)),
        pl.BlockSpec((16, 32), lambda p, c: (0, 0)),
    ]
    out_specs = [pl.BlockSpec((CH, 16), lambda p, c: (c, 0)),
                 pl.BlockSpec((CH, 16), lambda p, c: (c, 0))]
    out_shape = [jax.ShapeDtypeStruct((NN, 16), jnp.float32),
                 jax.ShapeDtypeStruct((NN, 16), jnp.float32)]
    args = [aggcat, aggcat, w_a, w_b]
    if last:
        in_specs += [pl.BlockSpec((CH, 3), lambda p, c: (c, 0)),
                     pl.BlockSpec((CH, 1), lambda p, c: (c, 0))]
        out_specs.append(pl.BlockSpec((CH, 1), lambda p, c: (c, 0)))
        out_shape.append(jax.ShapeDtypeStruct((NN, 1), jnp.int32))
        args += [pos2d, batch2d]
    return pl.pallas_call(
        body,
        grid=(2, NCCH),
        compiler_params=pltpu.CompilerParams(vmem_limit_bytes=100 * 2**20),
        in_specs=in_specs,
        out_specs=out_specs,
        out_shape=out_shape,
        scratch_shapes=[pltpu.VMEM((1, 32), jnp.float32),
                        pltpu.VMEM((1, 32), jnp.float32)],
    )(*args)


def _dense_final(plo4, phi4, wlo, whi):
    def body(a_ref, b_ref, wa_ref, wb_ref, o_ref):
        o_ref[...] = (jnp.dot(a_ref[...], wa_ref[...],
                              preferred_element_type=jnp.float32) +
                      jnp.dot(b_ref[...], wb_ref[...],
                              preferred_element_type=jnp.float32))

    return pl.pallas_call(
        body,
        out_shape=jax.ShapeDtypeStruct((NB, NOUT), jnp.float32),
    )(plo4, phi4, wlo, whi)


def kernel(x, pos, edge_index, batch, W1, W2, W3, W4, Wfc):
    src = edge_index[0]
    dst = edge_index[1]

    agg1 = _sc_agg_f1(x.reshape(NN), src, dst)
    h1 = _dense1(agg1.reshape(2 * NN, 1), W1)

    agg2 = _sc_agg_f16(h1, src, dst)
    h2lo, h2hi = _dense_mm(agg2, W2, W2)

    agg3 = _sc_agg_f32(h2lo, h2hi, src, dst)
    h3lo, h3hi = _dense_mm(agg3, W3[:16], W3[16:])

    agg4 = _sc_agg_f32(h3lo, h3hi, src, dst)
    h4lo, h4hi, idxq = _dense_mm(agg4, W4[:16], W4[16:],
                                 pos2d=pos, batch2d=batch.reshape(NN, 1))

    plo, phi = _sc_pool(h4lo, h4hi, idxq.reshape(NN))

    wr = Wfc.reshape(NOUT, NGRID, FPOOL)
    wlo = wr[:, :, :16].reshape(NOUT, NGRID * 16).T
    whi = wr[:, :, 16:].reshape(NOUT, NGRID * 16).T
    out = _dense_final(plo.reshape(NB, NGRID * 16),
                       phi.reshape(NB, NGRID * 16), wlo, whi)
    return out
